# baseline (speedup 1.0000x reference)
import dataclasses
import functools

import jax
import jax.numpy as jnp
from jax.experimental import pallas as pl
from jax.experimental.pallas import tpu as pltpu
from jax.experimental.pallas import tpu_sc as plsc

_K = 128
_NSUB = 16
_NCORE = 2
_RZ = 64


def _agg2(h2, src, dst, wn, V):
    R, F = h2.shape
    E = src.shape[0]
    K = 4096 // F
    RZ = min(128, max(32, 8192 // F))
    NBUF = 4
    src_pair = jnp.stack([src, src + V]).reshape(2 * (E // K), K)
    dst2 = dst.reshape(-1, K)
    wn2 = wn.reshape(-1, K)
    NCH = (E // K) // _NSUB
    RPS = V // _NSUB
    assert NCH % NBUF == 0 and NCH * _NSUB * K == E
    assert RPS * _NSUB == V and RPS % RZ == 0 and R == 2 * V and F % 16 == 0

    mesh = plsc.VectorSubcoreMesh(core_axis_name="core", subcore_axis_name="subcore")
    cp = pltpu.CompilerParams()
    if "needs_layout_passes" in pltpu.CompilerParams.__dataclass_fields__:
        cp = dataclasses.replace(cp, needs_layout_passes=False)
    if "use_tc_tiling_on_sc" in pltpu.CompilerParams.__dataclass_fields__:
        cp = dataclasses.replace(cp, use_tc_tiling_on_sc=False)

    @functools.partial(
        pl.kernel,
        out_type=[jax.ShapeDtypeStruct((R, F), jnp.float32),
                  jax.ShapeDtypeStruct((R, F), jnp.float32)],
        mesh=mesh,
        compiler_params=cp,
        scratch_types=(
            [pltpu.VMEM((K,), jnp.int32) for _ in range(NBUF)]
            + [pltpu.VMEM((K,), jnp.int32) for _ in range(NBUF)]
            + [pltpu.VMEM((K,), jnp.float32) for _ in range(NBUF)]
            + [pltpu.VMEM((K, F), jnp.float32) for _ in range(NBUF)]
            + [pltpu.VMEM((K, F), jnp.float32) for _ in range(NBUF)]
            + [
                pltpu.VMEM((RZ, F), jnp.float32),
                pltpu.VMEM_SHARED((V, F), jnp.float32),
            ]
            + [pltpu.SemaphoreType.DMA] * (3 * NBUF + 1)
        ),
    )
    def k(h_hbm, src_hbm, dst_hbm, wn_hbm, a1_hbm, b2_hbm, *rest):
        srcb = rest[0:NBUF]
        dstb = rest[NBUF:2 * NBUF]
        wnb = rest[2 * NBUF:3 * NBUF]
        bufs = rest[3 * NBUF:4 * NBUF]
        sbufs = rest[4 * NBUF:5 * NBUF]
        zbuf = rest[5 * NBUF]
        acc = rest[5 * NBUF + 1]
        isems = rest[5 * NBUF + 2:6 * NBUF + 2]
        gsems = rest[6 * NBUF + 2:7 * NBUF + 2]
        ssems = rest[7 * NBUF + 2:8 * NBUF + 2]
        sem_z = rest[8 * NBUF + 2]
        c = jax.lax.axis_index("core")
        s = jax.lax.axis_index("subcore")
        NZ = RPS // RZ
        srow0 = c * (E // K) + s * NCH
        row0 = s * NCH

        zv = jnp.zeros((16,), jnp.float32)

        @pl.loop(0, RZ)
        def _(r):
            for f in range(F // 16):
                zbuf[r, pl.ds(f * 16, 16)] = zv

        def start_idx(b, j):
            pltpu.async_copy(src_hbm.at[srow0 + j], srcb[b], isems[b])
            pltpu.async_copy(dst_hbm.at[row0 + j], dstb[b], isems[b])
            pltpu.async_copy(wn_hbm.at[row0 + j], wnb[b], isems[b])

        def ready_gather(feat_hbm, b, j):
            pltpu.make_async_copy(src_hbm.at[srow0 + j], srcb[b], isems[b]).wait()
            pltpu.make_async_copy(dst_hbm.at[row0 + j], dstb[b], isems[b]).wait()
            pltpu.make_async_copy(wn_hbm.at[row0 + j], wnb[b], isems[b]).wait()
            pltpu.async_copy(feat_hbm.at[srcb[b]], bufs[b], gsems[b])

        def scale(b):
            buf, sbuf, wv = bufs[b], sbufs[b], wnb[b]

            @pl.loop(0, K, step=16)
            def _(kbase):
                wvec = wv[pl.ds(kbase, 16)]
                for dk in range(16):
                    wsc = jnp.full((16,), wvec[dk])
                    ke = kbase + dk
                    for f in range(F // 16):
                        sl = (ke, pl.ds(f * 16, 16))
                        sbuf[sl] = buf[sl] * wsc

        def run_pass(feat_hbm, out_hbm):
            @pl.loop(0, NZ)
            def _(z):
                pltpu.async_copy(zbuf, acc.at[pl.ds(s * RPS + z * RZ, RZ)], sem_z)

            for b in range(NBUF):
                start_idx(b, b)

            @pl.loop(0, NZ)
            def _(z):
                pltpu.make_async_copy(
                    zbuf, acc.at[pl.ds(s * RPS + z * RZ, RZ)], sem_z).wait()

            plsc.subcore_barrier()

            for b in range(NBUF):
                ready_gather(feat_hbm, b, b)

            def process(b, j):
                pltpu.make_async_copy(feat_hbm.at[srcb[b]], bufs[b], gsems[b]).wait()
                scale(b)
                pltpu.async_copy(sbufs[b], acc.at[dstb[b]], ssems[b], add=True)

                @pl.when(j + NBUF < NCH)
                def _():
                    pltpu.make_async_copy(sbufs[b], acc.at[dstb[b]], ssems[b]).wait()
                    start_idx(b, j + NBUF)
                    ready_gather(feat_hbm, b, j + NBUF)

            @pl.loop(0, NCH, step=NBUF)
            def _(j):
                for b in range(NBUF):
                    process(b, j + b)

            for b in range(NBUF):
                pltpu.make_async_copy(sbufs[b], acc.at[dstb[b]], ssems[b]).wait()

            plsc.subcore_barrier()

            pltpu.sync_copy(acc.at[pl.ds(s * RPS, RPS)],
                            out_hbm.at[pl.ds(c * V + s * RPS, RPS)])

        run_pass(h_hbm, a1_hbm)
        plsc.subcore_barrier()
        run_pass(a1_hbm, b2_hbm)

    return k(h2, src_pair, dst2, wn2)


def _dense(terms, bias, relu):
    R = terms[0][0].shape[0]
    Fout = terms[0][1].shape[1]
    n = len(terms)
    BR = min(2048, R)
    b2 = bias.reshape(1, Fout)

    def body(*refs):
        xrefs = refs[:n]
        wrefs = refs[n:2 * n]
        bref = refs[2 * n]
        oref = refs[2 * n + 1]
        acc = bref[...]
        for xr, wr in zip(xrefs, wrefs):
            acc = acc + jnp.dot(xr[...], wr[...], preferred_element_type=jnp.float32)
        if relu:
            acc = jnp.maximum(acc, 0.0)
        oref[...] = acc

    in_specs = (
        [pl.BlockSpec((BR, x.shape[1]), lambda i: (i, 0)) for x, _ in terms]
        + [pl.BlockSpec(w.shape, lambda i: (0, 0)) for _, w in terms]
        + [pl.BlockSpec((1, Fout), lambda i: (0, 0))]
    )
    return pl.pallas_call(
        body,
        grid=(R // BR,),
        in_specs=in_specs,
        out_specs=pl.BlockSpec((BR, Fout), lambda i: (i, 0)),
        out_shape=jax.ShapeDtypeStruct((R, Fout), jnp.float32),
    )(*([x for x, _ in terms] + [w for _, w in terms] + [b2]))


def _pool(h2):
    R, F = h2.shape
    Rh = R // 2
    x3 = h2.reshape(Rh, 2 * F)
    BR = min(2048, Rh)

    def body(xref, oref):
        v = xref[...]
        oref[...] = jnp.maximum(v[:, :F], v[:, F:])

    return pl.pallas_call(
        body,
        grid=(Rh // BR,),
        in_specs=[pl.BlockSpec((BR, 2 * F), lambda i: (i, 0))],
        out_specs=pl.BlockSpec((BR, F), lambda i: (i, 0)),
        out_shape=jax.ShapeDtypeStruct((Rh, F), jnp.float32),
    )(x3)


def _prep_graph(g):
    src = g['src'].astype(jnp.int32)
    dst = g['dst'].astype(jnp.int32)
    wn = g['wn'].astype(jnp.float32)
    return src, dst, wn


def _cheb3(x2, p, gp, V, relu, extra=None, extra_bias=None):
    src2, dst2, wn2 = gp
    a1, b2 = _agg2(x2, src2, dst2, wn2, V)
    W = p['W']
    terms = [(x2, W[0] - W[2]), (a1, -W[1]), (b2, 2.0 * W[2])]
    bias = p['b']
    if extra is not None:
        terms.append(extra)
    if extra_bias is not None:
        bias = bias + extra_bias
    return _dense(terms, bias, relu)


def _res_block(x2, p, gp, V):
    h1 = _cheb3(x2, p['conv1'], gp, V, relu=True)
    out = _cheb3(
        h1, p['conv2'], gp, V, relu=True,
        extra=(x2, p['shortcut']['W'][0]),
        extra_bias=p['shortcut']['b'],
    )
    return out


def kernel(x, params, graphs):
    B, V5, Fin = x.shape
    FP = 16
    xp = jnp.pad(x, ((0, 0), (0, 0), (0, FP - Fin))).reshape(B * V5, FP)

    gps = [_prep_graph(g) for g in graphs]
    sizes = [g[0].shape[0] // 8 for g in gps]

    pc = params['conv']
    Wp = jnp.pad(pc['W'], ((0, 0), (0, FP - Fin), (0, 0)))
    h = _cheb3(xp, {'W': Wp, 'b': pc['b']}, gps[0], sizes[0], relu=True)

    e5 = _res_block(h, params['block5'], gps[0], sizes[0])
    e4 = _res_block(_pool(e5), params['block4'], gps[1], sizes[1])
    e3 = _res_block(_pool(e4), params['block3'], gps[2], sizes[2])
    e2 = _res_block(_pool(e3), params['block2'], gps[3], sizes[3])
    e1 = _res_block(_pool(e2), params['block1'], gps[4], sizes[4])
    e0 = _res_block(_pool(e1), params['block0'], gps[5], sizes[5])

    outs = (e0, e1, e2, e3, e4, e5)
    return tuple(o.reshape(B, o.shape[0] // B, o.shape[1]) for o in outs)


# speedup vs baseline: 1.1833x; 1.1833x over previous
import dataclasses
import functools

import jax
import jax.numpy as jnp
from jax.experimental import pallas as pl
from jax.experimental.pallas import tpu as pltpu
from jax.experimental.pallas import tpu_sc as plsc

_K = 128
_NSUB = 16
_NCORE = 2
_RZ = 64


def _agg2(h2, src, dst, wn, V):
    R, F = h2.shape
    E = src.shape[0]
    K = 8192 // F
    RZ = min(128, max(32, 8192 // F))
    NBUF = 2
    src_pair = jnp.stack([src, src + V]).reshape(2 * (E // K), K)
    dst2 = dst.reshape(-1, K)
    wn2 = wn.reshape(-1, K)
    NCH = (E // K) // _NSUB
    RPS = V // _NSUB
    assert NCH % NBUF == 0 and NCH * _NSUB * K == E
    assert RPS * _NSUB == V and RPS % RZ == 0 and R == 2 * V and F % 16 == 0

    mesh = plsc.VectorSubcoreMesh(core_axis_name="core", subcore_axis_name="subcore")
    cp = pltpu.CompilerParams()
    if "needs_layout_passes" in pltpu.CompilerParams.__dataclass_fields__:
        cp = dataclasses.replace(cp, needs_layout_passes=False)
    if "use_tc_tiling_on_sc" in pltpu.CompilerParams.__dataclass_fields__:
        cp = dataclasses.replace(cp, use_tc_tiling_on_sc=False)

    @functools.partial(
        pl.kernel,
        out_type=[jax.ShapeDtypeStruct((R, F), jnp.float32),
                  jax.ShapeDtypeStruct((R, F), jnp.float32)],
        mesh=mesh,
        compiler_params=cp,
        scratch_types=(
            [pltpu.VMEM((K,), jnp.int32) for _ in range(NBUF)]
            + [pltpu.VMEM((K,), jnp.int32) for _ in range(NBUF)]
            + [pltpu.VMEM((K,), jnp.float32) for _ in range(NBUF)]
            + [pltpu.VMEM((K, F), jnp.float32) for _ in range(NBUF)]
            + [pltpu.VMEM((K, F), jnp.float32) for _ in range(NBUF)]
            + [
                pltpu.VMEM((RZ, F), jnp.float32),
                pltpu.VMEM_SHARED((V, F), jnp.float32),
            ]
            + [pltpu.SemaphoreType.DMA] * (3 * NBUF + 1)
        ),
    )
    def k(h_hbm, src_hbm, dst_hbm, wn_hbm, a1_hbm, b2_hbm, *rest):
        srcb = rest[0:NBUF]
        dstb = rest[NBUF:2 * NBUF]
        wnb = rest[2 * NBUF:3 * NBUF]
        bufs = rest[3 * NBUF:4 * NBUF]
        sbufs = rest[4 * NBUF:5 * NBUF]
        zbuf = rest[5 * NBUF]
        acc = rest[5 * NBUF + 1]
        isems = rest[5 * NBUF + 2:6 * NBUF + 2]
        gsems = rest[6 * NBUF + 2:7 * NBUF + 2]
        ssems = rest[7 * NBUF + 2:8 * NBUF + 2]
        sem_z = rest[8 * NBUF + 2]
        c = jax.lax.axis_index("core")
        s = jax.lax.axis_index("subcore")
        NZ = RPS // RZ
        srow0 = c * (E // K) + s * NCH
        row0 = s * NCH

        zv = jnp.zeros((16,), jnp.float32)

        @pl.loop(0, RZ)
        def _(r):
            for f in range(F // 16):
                zbuf[r, pl.ds(f * 16, 16)] = zv

        def start_idx(b, j):
            pltpu.async_copy(src_hbm.at[srow0 + j], srcb[b], isems[b])
            pltpu.async_copy(dst_hbm.at[row0 + j], dstb[b], isems[b])
            pltpu.async_copy(wn_hbm.at[row0 + j], wnb[b], isems[b])

        def ready_gather(feat_hbm, b, j):
            pltpu.make_async_copy(src_hbm.at[srow0 + j], srcb[b], isems[b]).wait()
            pltpu.make_async_copy(dst_hbm.at[row0 + j], dstb[b], isems[b]).wait()
            pltpu.make_async_copy(wn_hbm.at[row0 + j], wnb[b], isems[b]).wait()
            pltpu.async_copy(feat_hbm.at[srcb[b]], bufs[b], gsems[b])

        def scale(b):
            buf, sbuf, wv = bufs[b], sbufs[b], wnb[b]

            @pl.loop(0, K, step=16)
            def _(kbase):
                wvec = wv[pl.ds(kbase, 16)]
                for dk in range(16):
                    wsc = jnp.full((16,), wvec[dk])
                    ke = kbase + dk
                    for f in range(F // 16):
                        sl = (ke, pl.ds(f * 16, 16))
                        sbuf[sl] = buf[sl] * wsc

        def run_pass(feat_hbm, out_hbm):
            @pl.loop(0, NZ)
            def _(z):
                pltpu.async_copy(zbuf, acc.at[pl.ds(s * RPS + z * RZ, RZ)], sem_z)

            for b in range(NBUF):
                start_idx(b, b)

            @pl.loop(0, NZ)
            def _(z):
                pltpu.make_async_copy(
                    zbuf, acc.at[pl.ds(s * RPS + z * RZ, RZ)], sem_z).wait()

            plsc.subcore_barrier()

            for b in range(NBUF):
                ready_gather(feat_hbm, b, b)

            def process(b, j):
                pltpu.make_async_copy(feat_hbm.at[srcb[b]], bufs[b], gsems[b]).wait()
                scale(b)
                pltpu.async_copy(sbufs[b], acc.at[dstb[b]], ssems[b], add=True)

                @pl.when(j + NBUF < NCH)
                def _():
                    pltpu.make_async_copy(sbufs[b], acc.at[dstb[b]], ssems[b]).wait()
                    start_idx(b, j + NBUF)
                    ready_gather(feat_hbm, b, j + NBUF)

            @pl.loop(0, NCH, step=NBUF)
            def _(j):
                for b in range(NBUF):
                    process(b, j + b)

            for b in range(NBUF):
                pltpu.make_async_copy(sbufs[b], acc.at[dstb[b]], ssems[b]).wait()

            plsc.subcore_barrier()

            pltpu.sync_copy(acc.at[pl.ds(s * RPS, RPS)],
                            out_hbm.at[pl.ds(c * V + s * RPS, RPS)])

        run_pass(h_hbm, a1_hbm)
        plsc.subcore_barrier()
        run_pass(a1_hbm, b2_hbm)

    return k(h2, src_pair, dst2, wn2)


def _dense(terms, bias, relu):
    R = terms[0][0].shape[0]
    Fout = terms[0][1].shape[1]
    n = len(terms)
    BR = min(2048, R)
    b2 = bias.reshape(1, Fout)

    def body(*refs):
        xrefs = refs[:n]
        wrefs = refs[n:2 * n]
        bref = refs[2 * n]
        oref = refs[2 * n + 1]
        acc = bref[...]
        for xr, wr in zip(xrefs, wrefs):
            acc = acc + jnp.dot(xr[...], wr[...], preferred_element_type=jnp.float32)
        if relu:
            acc = jnp.maximum(acc, 0.0)
        oref[...] = acc

    in_specs = (
        [pl.BlockSpec((BR, x.shape[1]), lambda i: (i, 0)) for x, _ in terms]
        + [pl.BlockSpec(w.shape, lambda i: (0, 0)) for _, w in terms]
        + [pl.BlockSpec((1, Fout), lambda i: (0, 0))]
    )
    return pl.pallas_call(
        body,
        grid=(R // BR,),
        in_specs=in_specs,
        out_specs=pl.BlockSpec((BR, Fout), lambda i: (i, 0)),
        out_shape=jax.ShapeDtypeStruct((R, Fout), jnp.float32),
    )(*([x for x, _ in terms] + [w for _, w in terms] + [b2]))


def _pool(h2):
    R, F = h2.shape
    Rh = R // 2
    x3 = h2.reshape(Rh, 2 * F)
    BR = min(2048, Rh)

    def body(xref, oref):
        v = xref[...]
        oref[...] = jnp.maximum(v[:, :F], v[:, F:])

    return pl.pallas_call(
        body,
        grid=(Rh // BR,),
        in_specs=[pl.BlockSpec((BR, 2 * F), lambda i: (i, 0))],
        out_specs=pl.BlockSpec((BR, F), lambda i: (i, 0)),
        out_shape=jax.ShapeDtypeStruct((Rh, F), jnp.float32),
    )(x3)


def _prep_graph(g):
    src = g['src'].astype(jnp.int32)
    dst = g['dst'].astype(jnp.int32)
    wn = g['wn'].astype(jnp.float32)
    return src, dst, wn


def _cheb3(x2, p, gp, V, relu, extra=None, extra_bias=None):
    src2, dst2, wn2 = gp
    a1, b2 = _agg2(x2, src2, dst2, wn2, V)
    W = p['W']
    terms = [(x2, W[0] - W[2]), (a1, -W[1]), (b2, 2.0 * W[2])]
    bias = p['b']
    if extra is not None:
        terms.append(extra)
    if extra_bias is not None:
        bias = bias + extra_bias
    return _dense(terms, bias, relu)


def _res_block(x2, p, gp, V):
    h1 = _cheb3(x2, p['conv1'], gp, V, relu=True)
    out = _cheb3(
        h1, p['conv2'], gp, V, relu=True,
        extra=(x2, p['shortcut']['W'][0]),
        extra_bias=p['shortcut']['b'],
    )
    return out


def kernel(x, params, graphs):
    B, V5, Fin = x.shape
    FP = 16
    xp = jnp.pad(x, ((0, 0), (0, 0), (0, FP - Fin))).reshape(B * V5, FP)

    gps = [_prep_graph(g) for g in graphs]
    sizes = [g[0].shape[0] // 8 for g in gps]

    pc = params['conv']
    Wp = jnp.pad(pc['W'], ((0, 0), (0, FP - Fin), (0, 0)))
    h = _cheb3(xp, {'W': Wp, 'b': pc['b']}, gps[0], sizes[0], relu=True)

    e5 = _res_block(h, params['block5'], gps[0], sizes[0])
    e4 = _res_block(_pool(e5), params['block4'], gps[1], sizes[1])
    e3 = _res_block(_pool(e4), params['block3'], gps[2], sizes[2])
    e2 = _res_block(_pool(e3), params['block2'], gps[3], sizes[3])
    e1 = _res_block(_pool(e2), params['block1'], gps[4], sizes[4])
    e0 = _res_block(_pool(e1), params['block0'], gps[5], sizes[5])

    outs = (e0, e1, e2, e3, e4, e5)
    return tuple(o.reshape(B, o.shape[0] // B, o.shape[1]) for o in outs)


# speedup vs baseline: 1.3852x; 1.1706x over previous
import dataclasses
import functools

import jax
import jax.numpy as jnp
from jax.experimental import pallas as pl
from jax.experimental.pallas import tpu as pltpu
from jax.experimental.pallas import tpu_sc as plsc

_K = 128
_NSUB = 16
_NCORE = 2
_RZ = 64


def _agg2(h2, src, dst, wn, V):
    R, F = h2.shape
    E = src.shape[0]
    K = 8192 // F
    RZ = min(128, max(32, 8192 // F))
    NIB = 4
    NRB = 2
    src_pair = jnp.stack([src, src + V]).reshape(2 * (E // K), K)
    dst2 = dst.reshape(-1, K)
    wn2 = wn.reshape(-1, K)
    NCH = (E // K) // _NSUB
    RPS = V // _NSUB
    assert NCH % NIB == 0 and NCH * _NSUB * K == E
    assert RPS * _NSUB == V and RPS % RZ == 0 and R == 2 * V and F % 16 == 0

    mesh = plsc.VectorSubcoreMesh(core_axis_name="core", subcore_axis_name="subcore")
    cp = pltpu.CompilerParams()
    if "needs_layout_passes" in pltpu.CompilerParams.__dataclass_fields__:
        cp = dataclasses.replace(cp, needs_layout_passes=False)
    if "use_tc_tiling_on_sc" in pltpu.CompilerParams.__dataclass_fields__:
        cp = dataclasses.replace(cp, use_tc_tiling_on_sc=False)

    @functools.partial(
        pl.kernel,
        out_type=[jax.ShapeDtypeStruct((R, F), jnp.float32),
                  jax.ShapeDtypeStruct((R, F), jnp.float32)],
        mesh=mesh,
        compiler_params=cp,
        scratch_types=(
            [pltpu.VMEM((K,), jnp.int32) for _ in range(NIB)]
            + [pltpu.VMEM((K,), jnp.int32) for _ in range(NIB)]
            + [pltpu.VMEM((K,), jnp.float32) for _ in range(NIB)]
            + [pltpu.VMEM((K, F), jnp.float32) for _ in range(NRB)]
            + [pltpu.VMEM((K, F), jnp.float32) for _ in range(NRB)]
            + [
                pltpu.VMEM((RZ, F), jnp.float32),
                pltpu.VMEM_SHARED((V, F), jnp.float32),
            ]
            + [pltpu.SemaphoreType.DMA] * (NIB + 2 * NRB + 1)
        ),
    )
    def k(h_hbm, src_hbm, dst_hbm, wn_hbm, a1_hbm, b2_hbm, *rest):
        srcb = rest[0:NIB]
        dstb = rest[NIB:2 * NIB]
        wnb = rest[2 * NIB:3 * NIB]
        bufs = rest[3 * NIB:3 * NIB + NRB]
        sbufs = rest[3 * NIB + NRB:3 * NIB + 2 * NRB]
        zbuf = rest[3 * NIB + 2 * NRB]
        acc = rest[3 * NIB + 2 * NRB + 1]
        nsem0 = 3 * NIB + 2 * NRB + 2
        isems = rest[nsem0:nsem0 + NIB]
        gsems = rest[nsem0 + NIB:nsem0 + NIB + NRB]
        ssems = rest[nsem0 + NIB + NRB:nsem0 + NIB + 2 * NRB]
        sem_z = rest[nsem0 + NIB + 2 * NRB]
        c = jax.lax.axis_index("core")
        s = jax.lax.axis_index("subcore")
        NZ = RPS // RZ
        srow0 = c * (E // K) + s * NCH
        row0 = s * NCH

        zv = jnp.zeros((16,), jnp.float32)

        @pl.loop(0, RZ)
        def _(r):
            for f in range(F // 16):
                zbuf[r, pl.ds(f * 16, 16)] = zv

        def start_idx(b, j):
            pltpu.async_copy(src_hbm.at[srow0 + j], srcb[b], isems[b])
            pltpu.async_copy(dst_hbm.at[row0 + j], dstb[b], isems[b])
            pltpu.async_copy(wn_hbm.at[row0 + j], wnb[b], isems[b])

        def wait_idx(b, j):
            pltpu.make_async_copy(src_hbm.at[srow0 + j], srcb[b], isems[b]).wait()
            pltpu.make_async_copy(dst_hbm.at[row0 + j], dstb[b], isems[b]).wait()
            pltpu.make_async_copy(wn_hbm.at[row0 + j], wnb[b], isems[b]).wait()

        def scale(rb, ib):
            buf, sbuf, wv = bufs[rb], sbufs[rb], wnb[ib]

            @pl.loop(0, K, step=16)
            def _(kbase):
                wvec = wv[pl.ds(kbase, 16)]
                for dk in range(16):
                    wsc = jnp.full((16,), wvec[dk])
                    ke = kbase + dk
                    for f in range(F // 16):
                        sl = (ke, pl.ds(f * 16, 16))
                        sbuf[sl] = buf[sl] * wsc

        def run_pass(feat_hbm, out_hbm):
            @pl.loop(0, NZ)
            def _(z):
                pltpu.async_copy(zbuf, acc.at[pl.ds(s * RPS + z * RZ, RZ)], sem_z)

            for b in range(NRB):
                start_idx(b, b)

            @pl.loop(0, NZ)
            def _(z):
                pltpu.make_async_copy(
                    zbuf, acc.at[pl.ds(s * RPS + z * RZ, RZ)], sem_z).wait()

            plsc.subcore_barrier()

            for b in range(NRB):
                wait_idx(b, b)
                pltpu.async_copy(feat_hbm.at[srcb[b]], bufs[b], gsems[b])

            @pl.loop(0, NCH, step=NIB)
            def _(j):
                for d in range(NIB):
                    ib = d
                    rb = d % NRB
                    ib2 = (d + 2) % NIB
                    jj = j + d
                    pltpu.make_async_copy(
                        feat_hbm.at[srcb[ib]], bufs[rb], gsems[rb]).wait()

                    @pl.when(jj >= 2)
                    def _():
                        pltpu.make_async_copy(
                            sbufs[rb], acc.at[dstb[ib2]], ssems[rb]).wait()

                    @pl.when(jj + 2 < NCH)
                    def _():
                        start_idx(ib2, jj + 2)

                    scale(rb, ib)
                    pltpu.async_copy(sbufs[rb], acc.at[dstb[ib]], ssems[rb],
                                     add=True)

                    @pl.when(jj + 2 < NCH)
                    def _():
                        wait_idx(ib2, jj + 2)
                        pltpu.async_copy(feat_hbm.at[srcb[ib2]], bufs[rb],
                                         gsems[rb])

            pltpu.make_async_copy(sbufs[0], acc.at[dstb[2]], ssems[0]).wait()
            pltpu.make_async_copy(sbufs[1], acc.at[dstb[3]], ssems[1]).wait()

            plsc.subcore_barrier()

            pltpu.sync_copy(acc.at[pl.ds(s * RPS, RPS)],
                            out_hbm.at[pl.ds(c * V + s * RPS, RPS)])

        run_pass(h_hbm, a1_hbm)
        plsc.subcore_barrier()
        run_pass(a1_hbm, b2_hbm)

    return k(h2, src_pair, dst2, wn2)


def _dense(terms, bias, relu):
    R = terms[0][0].shape[0]
    Fout = terms[0][1].shape[1]
    n = len(terms)
    BR = min(2048, R)
    b2 = bias.reshape(1, Fout)

    def body(*refs):
        xrefs = refs[:n]
        wrefs = refs[n:2 * n]
        bref = refs[2 * n]
        oref = refs[2 * n + 1]
        acc = bref[...]
        for xr, wr in zip(xrefs, wrefs):
            acc = acc + jnp.dot(xr[...], wr[...], preferred_element_type=jnp.float32)
        if relu:
            acc = jnp.maximum(acc, 0.0)
        oref[...] = acc

    in_specs = (
        [pl.BlockSpec((BR, x.shape[1]), lambda i: (i, 0)) for x, _ in terms]
        + [pl.BlockSpec(w.shape, lambda i: (0, 0)) for _, w in terms]
        + [pl.BlockSpec((1, Fout), lambda i: (0, 0))]
    )
    return pl.pallas_call(
        body,
        grid=(R // BR,),
        in_specs=in_specs,
        out_specs=pl.BlockSpec((BR, Fout), lambda i: (i, 0)),
        out_shape=jax.ShapeDtypeStruct((R, Fout), jnp.float32),
    )(*([x for x, _ in terms] + [w for _, w in terms] + [b2]))


def _pool(h2):
    R, F = h2.shape
    Rh = R // 2
    x3 = h2.reshape(Rh, 2 * F)
    BR = min(2048, Rh)

    def body(xref, oref):
        v = xref[...]
        oref[...] = jnp.maximum(v[:, :F], v[:, F:])

    return pl.pallas_call(
        body,
        grid=(Rh // BR,),
        in_specs=[pl.BlockSpec((BR, 2 * F), lambda i: (i, 0))],
        out_specs=pl.BlockSpec((BR, F), lambda i: (i, 0)),
        out_shape=jax.ShapeDtypeStruct((Rh, F), jnp.float32),
    )(x3)


def _prep_graph(g):
    src = g['src'].astype(jnp.int32)
    dst = g['dst'].astype(jnp.int32)
    wn = g['wn'].astype(jnp.float32)
    return src, dst, wn


def _cheb3(x2, p, gp, V, relu, extra=None, extra_bias=None):
    src2, dst2, wn2 = gp
    a1, b2 = _agg2(x2, src2, dst2, wn2, V)
    W = p['W']
    terms = [(x2, W[0] - W[2]), (a1, -W[1]), (b2, 2.0 * W[2])]
    bias = p['b']
    if extra is not None:
        terms.append(extra)
    if extra_bias is not None:
        bias = bias + extra_bias
    return _dense(terms, bias, relu)


def _res_block(x2, p, gp, V):
    h1 = _cheb3(x2, p['conv1'], gp, V, relu=True)
    out = _cheb3(
        h1, p['conv2'], gp, V, relu=True,
        extra=(x2, p['shortcut']['W'][0]),
        extra_bias=p['shortcut']['b'],
    )
    return out


def kernel(x, params, graphs):
    B, V5, Fin = x.shape
    FP = 16
    xp = jnp.pad(x, ((0, 0), (0, 0), (0, FP - Fin))).reshape(B * V5, FP)

    gps = [_prep_graph(g) for g in graphs]
    sizes = [g[0].shape[0] // 8 for g in gps]

    pc = params['conv']
    Wp = jnp.pad(pc['W'], ((0, 0), (0, FP - Fin), (0, 0)))
    h = _cheb3(xp, {'W': Wp, 'b': pc['b']}, gps[0], sizes[0], relu=True)

    e5 = _res_block(h, params['block5'], gps[0], sizes[0])
    e4 = _res_block(_pool(e5), params['block4'], gps[1], sizes[1])
    e3 = _res_block(_pool(e4), params['block3'], gps[2], sizes[2])
    e2 = _res_block(_pool(e3), params['block2'], gps[3], sizes[3])
    e1 = _res_block(_pool(e2), params['block1'], gps[4], sizes[4])
    e0 = _res_block(_pool(e1), params['block0'], gps[5], sizes[5])

    outs = (e0, e1, e2, e3, e4, e5)
    return tuple(o.reshape(B, o.shape[0] // B, o.shape[1]) for o in outs)
